# baseline (speedup 1.0000x reference)
import functools

import jax
import jax.numpy as jnp
from jax import lax
from jax.experimental import pallas as pl
from jax.experimental.pallas import tpu as pltpu
from jax.experimental.pallas import tpu_sc as plsc

HIDDEN = 2048
NUM_EXPERTS = 64
TOP_K = 8
TOKENS = 16384

TM = 2048

_NC = 2
_NS = 16
_NW = _NC * _NS
_TPW = TOKENS // _NW
_GRP = 16
_NG = _TPW // _GRP


def _router_tc_body(x_ref, w_ref, scores_ref, scores_t_ref):
    x = x_ref[...]
    w = w_ref[...]
    logits = lax.dot_general(
        x, w, (((1,), (1,)), ((), ())),
        preferred_element_type=jnp.float32)
    m = jnp.max(logits, axis=-1, keepdims=True)
    e = jnp.exp(logits - m)
    scores = e / jnp.sum(e, axis=-1, keepdims=True)
    scores_ref[...] = scores
    st = jnp.swapaxes(scores.reshape(TM // _TPW, _TPW, NUM_EXPERTS), 1, 2)
    scores_t_ref[...] = st.reshape(TM * NUM_EXPERTS // 128, 128)


def _scores_tc(x, W):
    n_tokens = x.shape[0]
    return pl.pallas_call(
        _router_tc_body,
        grid=(n_tokens // TM,),
        in_specs=[
            pl.BlockSpec((TM, HIDDEN), lambda i: (i, 0)),
            pl.BlockSpec((NUM_EXPERTS, HIDDEN), lambda i: (0, 0)),
        ],
        out_specs=[
            pl.BlockSpec((TM, NUM_EXPERTS), lambda i: (i, 0)),
            pl.BlockSpec((TM * NUM_EXPERTS // 128, 128), lambda i: (i, 0)),
        ],
        out_shape=[
            jax.ShapeDtypeStruct((n_tokens, NUM_EXPERTS), jnp.float32),
            jax.ShapeDtypeStruct((n_tokens * NUM_EXPERTS // 128, 128),
                                 jnp.float32),
        ],
        compiler_params=pltpu.CompilerParams(
            dimension_semantics=("arbitrary",),
        ),
    )(x, W)


def _topk_sc_body(scores_t_hbm, wts_hbm, idx_hbm, buf, wts_v, idx_v):
    wid = lax.axis_index("s") * _NC + lax.axis_index("c")
    base = wid * _TPW
    rows_per_w = _TPW * NUM_EXPERTS // 128
    pltpu.sync_copy(scores_t_hbm.at[pl.ds(wid * rows_per_w, rows_per_w), :],
                    buf)

    lanes = lax.iota(jnp.int32, _GRP)

    def _tree_max(vals):
        while len(vals) > 1:
            vals = [jnp.maximum(vals[i], vals[i + 1])
                    for i in range(0, len(vals) - 1, 2)] + (
                        [vals[-1]] if len(vals) % 2 else [])
        return vals[0]

    def _tree_min(vals):
        while len(vals) > 1:
            vals = [jnp.minimum(vals[i], vals[i + 1])
                    for i in range(0, len(vals) - 1, 2)] + (
                        [vals[-1]] if len(vals) % 2 else [])
        return vals[0]

    def _one_group(g):
        tok = g * _GRP + lanes
        s = []
        for j in range(4):
            cs = [buf[(16 * j + t) * (_TPW // 128) + g // 8,
                      pl.ds((g % 8) * _GRP, _GRP)]
                  for t in range(16)]
            s.append(_tree_max(cs))

        rem_lo = jnp.zeros((_GRP,), jnp.int32)
        rem_hi = jnp.zeros((_GRP,), jnp.int32)
        one = jnp.full((_GRP,), 1, jnp.int32)
        for k in range(TOP_K):
            m = _tree_max(list(s))
            jstar = jnp.full((_GRP,), 3, jnp.int32)
            for j in (2, 1, 0):
                jstar = jnp.where(s[j] == m, j, jstar)
            cbase = jstar * 16
            word = jnp.where(jstar < 2, rem_lo, rem_hi)
            shbase = jnp.bitwise_and(cbase, 31)
            fl = [(cbase + t) * _TPW + tok for t in range(16)]
            cs = [plsc.load_gather(
                buf, [jax.lax.shift_right_logical(fl[t], 7),
                      jnp.bitwise_and(fl[t], 127)]) for t in range(16)]
            csm = [jnp.where(
                jnp.bitwise_and(
                    jax.lax.shift_right_logical(word, shbase + t), 1) != 0,
                -1.0, cs[t]) for t in range(16)]
            tts = [jnp.where(csm[t] == m, t, NUM_EXPERTS) for t in range(16)]
            tstar = _tree_min(tts)
            estar = cbase + tstar
            wts_v[pl.ds(k * _TPW + g * _GRP, _GRP)] = m
            idx_v[pl.ds(k * _TPW + g * _GRP, _GRP)] = estar
            bit = jax.lax.shift_left(one, jnp.bitwise_and(estar, 31))
            is_lo = estar < 32
            rem_lo = jnp.bitwise_or(rem_lo, jnp.where(is_lo, bit, 0))
            rem_hi = jnp.bitwise_or(rem_hi, jnp.where(is_lo, 0, bit))
            csk = [jnp.where(tstar == t, -1.0, csm[t]) for t in range(16)]
            news = _tree_max(csk)
            for j in range(4):
                s[j] = jnp.where(jstar == j, news, s[j])

    def group_body(g, carry):
        _one_group(g)
        _one_group(g + _NG // 2)
        return carry

    lax.fori_loop(0, _NG // 2, group_body, 0)

    for k in range(TOP_K):
        pltpu.sync_copy(wts_v.at[pl.ds(k * _TPW, _TPW)],
                        wts_hbm.at[k, pl.ds(base, _TPW)])
        pltpu.sync_copy(idx_v.at[pl.ds(k * _TPW, _TPW)],
                        idx_hbm.at[k, pl.ds(base, _TPW)])


_topk_sc = functools.partial(
    pl.kernel,
    out_type=[
        jax.ShapeDtypeStruct((TOP_K, TOKENS), jnp.float32),
        jax.ShapeDtypeStruct((TOP_K, TOKENS), jnp.int32),
    ],
    mesh=plsc.VectorSubcoreMesh(
        core_axis_name="c", subcore_axis_name="s",
        num_cores=_NC, num_subcores=_NS),
    scratch_types=[
        pltpu.VMEM((NUM_EXPERTS * _TPW // 128, 128), jnp.float32),
        pltpu.VMEM((TOP_K * _TPW,), jnp.float32),
        pltpu.VMEM((TOP_K * _TPW,), jnp.int32),
    ],
    compiler_params=pltpu.CompilerParams(needs_layout_passes=False),
)(_topk_sc_body)


@jax.jit
def kernel(x, W):
    scores, scores_t = _scores_tc(x, W)
    wts_t, idx_t = _topk_sc(scores_t)
    return scores, wts_t.T, idx_t.T


# speedup vs baseline: 1.0006x; 1.0006x over previous
import functools

import jax
import jax.numpy as jnp
from jax import lax
from jax.experimental import pallas as pl
from jax.experimental.pallas import tpu as pltpu
from jax.experimental.pallas import tpu_sc as plsc

HIDDEN = 2048
NUM_EXPERTS = 64
TOP_K = 8
TOKENS = 16384

TM = 2048

_NC = 2
_NS = 16
_NW = _NC * _NS
_TPW = TOKENS // _NW
_GRP = 16
_NG = _TPW // _GRP


def _router_tc_body(x_ref, w_ref, scores_ref, scores_t_ref):
    x = x_ref[...]
    w = w_ref[...]
    logits = lax.dot_general(
        x, w, (((1,), (1,)), ((), ())),
        preferred_element_type=jnp.float32)
    m = jnp.max(logits, axis=-1, keepdims=True)
    e = jnp.exp(logits - m)
    scores = e / jnp.sum(e, axis=-1, keepdims=True)
    scores_ref[...] = scores
    st = jnp.swapaxes(scores.reshape(TM // _TPW, _TPW, NUM_EXPERTS), 1, 2)
    scores_t_ref[...] = st.reshape(TM * NUM_EXPERTS // 128, 128)


def _scores_tc(x, W):
    n_tokens = x.shape[0]
    return pl.pallas_call(
        _router_tc_body,
        grid=(n_tokens // TM,),
        in_specs=[
            pl.BlockSpec((TM, HIDDEN), lambda i: (i, 0)),
            pl.BlockSpec((NUM_EXPERTS, HIDDEN), lambda i: (0, 0)),
        ],
        out_specs=[
            pl.BlockSpec((TM, NUM_EXPERTS), lambda i: (i, 0)),
            pl.BlockSpec((TM * NUM_EXPERTS // 128, 128), lambda i: (i, 0)),
        ],
        out_shape=[
            jax.ShapeDtypeStruct((n_tokens, NUM_EXPERTS), jnp.float32),
            jax.ShapeDtypeStruct((16 * n_tokens * NUM_EXPERTS // 128, 128),
                                 jnp.float32),
        ],
        compiler_params=pltpu.CompilerParams(
            dimension_semantics=("arbitrary",),
        ),
    )(x, W)


def _topk_sc_body(scores_t_hbm, wts_hbm, idx_hbm, buf, wts_v, idx_v):
    wid = lax.axis_index("s") * _NC + lax.axis_index("c")
    base = wid * _TPW
    rows_per_w = _TPW * NUM_EXPERTS // 128
    pltpu.sync_copy(scores_t_hbm.at[pl.ds(wid * rows_per_w, rows_per_w), :],
                    buf)

    lanes = lax.iota(jnp.int32, _GRP)

    def _tree_max(vals):
        while len(vals) > 1:
            vals = [jnp.maximum(vals[i], vals[i + 1])
                    for i in range(0, len(vals) - 1, 2)] + (
                        [vals[-1]] if len(vals) % 2 else [])
        return vals[0]

    def _tree_min(vals):
        while len(vals) > 1:
            vals = [jnp.minimum(vals[i], vals[i + 1])
                    for i in range(0, len(vals) - 1, 2)] + (
                        [vals[-1]] if len(vals) % 2 else [])
        return vals[0]

    def _one_group(g):
        tok = g * _GRP + lanes
        s = []
        for j in range(4):
            cs = [buf[(16 * j + t) * (_TPW // 128) + g // 8,
                      pl.ds((g % 8) * _GRP, _GRP)]
                  for t in range(16)]
            s.append(_tree_max(cs))

        rem_lo = jnp.zeros((_GRP,), jnp.int32)
        rem_hi = jnp.zeros((_GRP,), jnp.int32)
        one = jnp.full((_GRP,), 1, jnp.int32)
        for k in range(TOP_K):
            m = _tree_max(list(s))
            jstar = jnp.full((_GRP,), 3, jnp.int32)
            for j in (2, 1, 0):
                jstar = jnp.where(s[j] == m, j, jstar)
            cbase = jstar * 16
            word = jnp.where(jstar < 2, rem_lo, rem_hi)
            shbase = jnp.bitwise_and(cbase, 31)
            fl = [(cbase + t) * _TPW + tok for t in range(16)]
            cs = [plsc.load_gather(
                buf, [jax.lax.shift_right_logical(fl[t], 7),
                      jnp.bitwise_and(fl[t], 127)]) for t in range(16)]
            csm = [jnp.where(
                jnp.bitwise_and(
                    jax.lax.shift_right_logical(word, shbase + t), 1) != 0,
                -1.0, cs[t]) for t in range(16)]
            tts = [jnp.where(csm[t] == m, t, NUM_EXPERTS) for t in range(16)]
            tstar = _tree_min(tts)
            estar = cbase + tstar
            wts_v[pl.ds(k * _TPW + g * _GRP, _GRP)] = m
            idx_v[pl.ds(k * _TPW + g * _GRP, _GRP)] = estar
            bit = jax.lax.shift_left(one, jnp.bitwise_and(estar, 31))
            is_lo = estar < 32
            rem_lo = jnp.bitwise_or(rem_lo, jnp.where(is_lo, bit, 0))
            rem_hi = jnp.bitwise_or(rem_hi, jnp.where(is_lo, 0, bit))
            csk = [jnp.where(tstar == t, -1.0, csm[t]) for t in range(16)]
            news = _tree_max(csk)
            for j in range(4):
                s[j] = jnp.where(jstar == j, news, s[j])

    def group_body(g, carry):
        _one_group(g)
        _one_group(g + _NG // 2)
        return carry

    lax.fori_loop(0, _NG // 2, group_body, 0)

    for k in range(TOP_K):
        pltpu.sync_copy(wts_v.at[pl.ds(k * _TPW, _TPW)],
                        wts_hbm.at[k, pl.ds(base, _TPW)])
        pltpu.sync_copy(idx_v.at[pl.ds(k * _TPW, _TPW)],
                        idx_hbm.at[k, pl.ds(base, _TPW)])


_topk_sc = functools.partial(
    pl.kernel,
    out_type=[
        jax.ShapeDtypeStruct((TOP_K, TOKENS), jnp.float32),
        jax.ShapeDtypeStruct((TOP_K, TOKENS), jnp.int32),
    ],
    mesh=plsc.VectorSubcoreMesh(
        core_axis_name="c", subcore_axis_name="s",
        num_cores=_NC, num_subcores=_NS),
    scratch_types=[
        pltpu.VMEM((NUM_EXPERTS * _TPW // 128, 128), jnp.float32),
        pltpu.VMEM((TOP_K * _TPW,), jnp.float32),
        pltpu.VMEM((TOP_K * _TPW,), jnp.int32),
    ],
    compiler_params=pltpu.CompilerParams(needs_layout_passes=False),
)(_topk_sc_body)


@jax.jit
def kernel(x, W):
    scores, scores_t = _scores_tc(x, W)
    wts_t, idx_t = _topk_sc(scores_t)
    return scores, wts_t.T, idx_t.T
